# baseline (speedup 1.0000x reference)
import jax
import jax.numpy as jnp
from jax.experimental import pallas as pl

_G = 256
_K = 32


def _fps(xyz, G):
    B, N, _ = xyz.shape

    def body(i, carry):
        distance, farthest, cent = carry
        cent = cent.at[:, i].set(farthest)
        c = jnp.take_along_axis(xyz, farthest[:, None, None], axis=1)
        d = jnp.sum((xyz - c) ** 2, axis=-1)
        distance = jnp.minimum(distance, d)
        farthest = jnp.argmax(distance, axis=-1).astype(jnp.int32)
        return (distance, farthest, cent)

    init = (jnp.full((B, N), jnp.inf, dtype=xyz.dtype),
            jnp.zeros((B,), dtype=jnp.int32),
            jnp.zeros((B, G), dtype=jnp.int32))
    _, _, cent = jax.lax.fori_loop(0, G, body, init)
    return cent


def _sqdist_explicit(a, b):
    a2 = jnp.sum(a * a, axis=-1)[:, :, None]
    b2 = jnp.sum(b * b, axis=-1)[:, None, :]
    return a2 + b2 - 2.0 * jnp.einsum('bmd,bnd->bmn', a, b)


def _morton_sort(center, G):
    B = center.shape[0]
    d = _sqdist_explicit(center, center)
    ar = jnp.arange(G)
    d = d.at[:, ar, ar].set(jnp.inf)
    d = d.at[:, :, 0].set(jnp.inf)

    def body(i, carry):
        dd, last, sel = carry
        row = jnp.take_along_axis(dd, last[:, None, None], axis=1)[:, 0, :]
        nxt = jnp.argmin(row, axis=-1).astype(jnp.int32)
        sel = sel.at[:, i].set(nxt)
        dd = dd.at[jnp.arange(B), :, nxt].set(jnp.inf)
        return (dd, nxt, sel)

    sel0 = jnp.zeros((B, G), dtype=jnp.int32)
    _, _, sel = jax.lax.fori_loop(1, G, body, (d, jnp.zeros((B,), jnp.int32), sel0))
    return (jnp.arange(B)[:, None] * G + sel).reshape(-1)


def _identity_pallas(x):
    def body(x_ref, o_ref):
        o_ref[...] = x_ref[...]
    return pl.pallas_call(
        body, out_shape=jax.ShapeDtypeStruct(x.shape, x.dtype))(x)


def kernel(x, xyz):
    B, N, C = x.shape
    G, K = _G, _K
    fps_idx = _fps(xyz, G)
    center = jnp.take_along_axis(xyz, fps_idx[:, :, None], axis=1)
    new_points = jnp.take_along_axis(x, fps_idx[:, :, None], axis=1)
    d = _sqdist_explicit(center, xyz)
    _, idx = jax.lax.top_k(-d, K)
    bidx = jnp.arange(B)[:, None, None]
    neighborhood = x[bidx, idx]
    neighborhood = neighborhood - new_points[:, :, None, :]
    rep = jnp.broadcast_to(new_points[:, :, None, :], (B, G, K, C))
    neighborhood = jnp.concatenate([neighborhood, rep], axis=-1)
    si = _morton_sort(center, G)
    neighborhood = neighborhood.reshape(B * G, K, 2 * C)[si].reshape(B, G, K, 2 * C)
    center = center.reshape(B * G, 3)[si].reshape(B, G, 3)
    center = _identity_pallas(center)
    return (neighborhood, center)


# speedup vs baseline: 1.5321x; 1.5321x over previous
import functools

import jax
import jax.numpy as jnp
from jax.experimental import pallas as pl
from jax.experimental.pallas import tpu as pltpu

_G = 256
_K = 32


def _fps_body(xt_ref, lane_ref, glane_ref, out_ref, *, B, N, G):
    xs = xt_ref[:, 0, :]
    ys = xt_ref[:, 1, :]
    zs = xt_ref[:, 2, :]
    lane = lane_ref[...]
    glane = glane_ref[...]

    def step(i, carry):
        dist, far, sel = carry
        sel = jnp.where(glane == i, far, sel)
        oh = lane == far
        cx = jnp.sum(jnp.where(oh, xs, 0.0), axis=1, keepdims=True)
        cy = jnp.sum(jnp.where(oh, ys, 0.0), axis=1, keepdims=True)
        cz = jnp.sum(jnp.where(oh, zs, 0.0), axis=1, keepdims=True)
        dx = xs - cx
        dy = ys - cy
        dz = zs - cz
        d = (dx * dx + dy * dy) + dz * dz
        dist = jnp.minimum(dist, d)
        m = jnp.max(dist, axis=1, keepdims=True)
        far_new = jnp.min(jnp.where(dist == m, lane, N), axis=1, keepdims=True)
        return dist, far_new, sel

    dist0 = jnp.full((B, N), jnp.inf, dtype=jnp.float32)
    far0 = jnp.zeros((B, 1), dtype=jnp.int32)
    sel0 = jnp.zeros((B, G), dtype=jnp.int32)
    _, _, sel = jax.lax.fori_loop(0, G, step, (dist0, far0, sel0))
    out_ref[...] = sel


def _fps_pallas(xyz):
    B, N, _ = xyz.shape
    xt = jnp.transpose(xyz, (0, 2, 1))
    lane = jnp.broadcast_to(jnp.arange(N, dtype=jnp.int32)[None, :], (B, N))
    glane = jnp.broadcast_to(jnp.arange(_G, dtype=jnp.int32)[None, :], (B, _G))
    return pl.pallas_call(
        functools.partial(_fps_body, B=B, N=N, G=_G),
        out_shape=jax.ShapeDtypeStruct((B, _G), jnp.int32),
    )(xt, lane, glane)


def _chain_body(d_ref, out_ref, *, B, G):
    lane = jax.lax.broadcasted_iota(jnp.int32, (1, G), 1)

    for b in range(B):
        def step(i, carry):
            visited, last, sel = carry
            row = d_ref[b, pl.ds(last, 1), :]
            row = jnp.where(visited != 0, jnp.inf, row)
            m = jnp.min(row, axis=1, keepdims=True)
            nxt = jnp.min(jnp.where(row == m, lane, G), axis=1, keepdims=True)
            nxt_s = jnp.sum(nxt)
            visited = visited | (lane == nxt_s).astype(jnp.int32)
            sel = jnp.where(lane == i, nxt_s, sel)
            return visited, nxt_s, sel

        visited0 = jnp.zeros((1, G), dtype=jnp.int32)
        sel0 = jnp.zeros((1, G), dtype=jnp.int32)
        _, _, sel = jax.lax.fori_loop(1, G, step,
                                      (visited0, jnp.int32(0), sel0))
        out_ref[b, :] = sel[0, :]


def _chain_pallas(dcc):
    B, G, _ = dcc.shape
    return pl.pallas_call(
        functools.partial(_chain_body, B=B, G=G),
        out_shape=jax.ShapeDtypeStruct((B, G), jnp.int32),
    )(dcc)


def _sqdist(a, b):
    a2 = jnp.sum(a * a, axis=-1)[:, :, None]
    b2 = jnp.sum(b * b, axis=-1)[:, None, :]
    return a2 + b2 - 2.0 * jnp.einsum('bmd,bnd->bmn', a, b)


def kernel(x, xyz):
    B, N, C = x.shape
    G, K = _G, _K

    fps_idx = _fps_pallas(xyz)
    center = jnp.take_along_axis(xyz, fps_idx[:, :, None], axis=1)
    new_points = jnp.take_along_axis(x, fps_idx[:, :, None], axis=1)

    d = _sqdist(center, xyz)
    _, idx = jax.lax.top_k(-d, K)
    bidx = jnp.arange(B)[:, None, None]
    neighborhood = x[bidx, idx]
    neighborhood = neighborhood - new_points[:, :, None, :]
    rep = jnp.broadcast_to(new_points[:, :, None, :], (B, G, K, C))
    neighborhood = jnp.concatenate([neighborhood, rep], axis=-1)

    dcc = _sqdist(center, center)
    ar = jnp.arange(G)
    dcc = dcc.at[:, ar, ar].set(jnp.inf)
    dcc = dcc.at[:, :, 0].set(jnp.inf)
    sel = _chain_pallas(dcc)
    si = (jnp.arange(B)[:, None] * G + sel).reshape(-1)

    neighborhood = neighborhood.reshape(B * G, K, 2 * C)[si].reshape(B, G, K, 2 * C)
    center = center.reshape(B * G, 3)[si].reshape(B, G, 3)
    return (neighborhood, center)


# speedup vs baseline: 10.0221x; 6.5416x over previous
import functools

import jax
import jax.numpy as jnp
from jax.experimental import pallas as pl
from jax.experimental.pallas import tpu as pltpu

_G = 256
_K = 32


def _fps_body(xt_ref, lane_ref, glane_ref, out_ref, *, B, N, G):
    xs = xt_ref[:, 0, :]
    ys = xt_ref[:, 1, :]
    zs = xt_ref[:, 2, :]
    lane = lane_ref[...]
    glane = glane_ref[...]

    def step(i, carry):
        dist, far, sel = carry
        sel = jnp.where(glane == i, far, sel)
        oh = lane == far
        cx = jnp.sum(jnp.where(oh, xs, 0.0), axis=1, keepdims=True)
        cy = jnp.sum(jnp.where(oh, ys, 0.0), axis=1, keepdims=True)
        cz = jnp.sum(jnp.where(oh, zs, 0.0), axis=1, keepdims=True)
        dx = xs - cx
        dy = ys - cy
        dz = zs - cz
        d = (dx * dx + dy * dy) + dz * dz
        dist = jnp.minimum(dist, d)
        m = jnp.max(dist, axis=1, keepdims=True)
        far_new = jnp.min(jnp.where(dist == m, lane, N), axis=1, keepdims=True)
        return dist, far_new, sel

    dist0 = jnp.full((B, N), jnp.inf, dtype=jnp.float32)
    far0 = jnp.zeros((B, 1), dtype=jnp.int32)
    sel0 = jnp.zeros((B, G), dtype=jnp.int32)
    _, _, sel = jax.lax.fori_loop(0, G, step, (dist0, far0, sel0))
    out_ref[...] = sel


def _fps_pallas(xyz):
    B, N, _ = xyz.shape
    xt = jnp.transpose(xyz, (0, 2, 1))
    lane = jnp.broadcast_to(jnp.arange(N, dtype=jnp.int32)[None, :], (B, N))
    glane = jnp.broadcast_to(jnp.arange(_G, dtype=jnp.int32)[None, :], (B, _G))
    return pl.pallas_call(
        functools.partial(_fps_body, B=B, N=N, G=_G),
        out_shape=jax.ShapeDtypeStruct((B, _G), jnp.int32),
    )(xt, lane, glane)


def _chain_body(d_ref, out_ref, *, B, G):
    lane = jax.lax.broadcasted_iota(jnp.int32, (1, G), 1)

    for b in range(B):
        def step(i, carry):
            visited, last, sel = carry
            row = d_ref[b, pl.ds(last, 1), :]
            row = jnp.where(visited != 0, jnp.inf, row)
            m = jnp.min(row, axis=1, keepdims=True)
            nxt = jnp.min(jnp.where(row == m, lane, G), axis=1, keepdims=True)
            nxt_s = jnp.sum(nxt)
            visited = visited | (lane == nxt_s).astype(jnp.int32)
            sel = jnp.where(lane == i, nxt_s, sel)
            return visited, nxt_s, sel

        visited0 = jnp.zeros((1, G), dtype=jnp.int32)
        sel0 = jnp.zeros((1, G), dtype=jnp.int32)
        _, _, sel = jax.lax.fori_loop(1, G, step,
                                      (visited0, jnp.int32(0), sel0))
        out_ref[b, :] = sel[0, :]


def _chain_pallas(dcc):
    B, G, _ = dcc.shape
    return pl.pallas_call(
        functools.partial(_chain_body, B=B, G=G),
        out_shape=jax.ShapeDtypeStruct((B, G), jnp.int32),
    )(dcc)


def _sqdist(a, b):
    a2 = jnp.sum(a * a, axis=-1)[:, :, None]
    b2 = jnp.sum(b * b, axis=-1)[:, None, :]
    return a2 + b2 - 2.0 * jnp.einsum('bmd,bnd->bmn', a, b)


def kernel(x, xyz):
    B, N, C = x.shape
    G, K = _G, _K

    fps_idx = _fps_pallas(xyz)
    center = jnp.take_along_axis(xyz, fps_idx[:, :, None], axis=1)
    new_points = jnp.take_along_axis(x, fps_idx[:, :, None], axis=1)

    d = _sqdist(center, xyz)
    d4 = d.reshape(B, G, N // 16, 16)
    m16 = jnp.min(d4, axis=-1)
    _, cid = jax.lax.top_k(-m16, K)
    cid = jnp.sort(cid, axis=-1)
    cand = jnp.take_along_axis(d4, cid[..., None], axis=2)
    _, p = jax.lax.top_k(-cand.reshape(B, G, 16 * K), K)
    cidx = jnp.take_along_axis(cid, p // 16, axis=-1)
    idx = cidx * 16 + (p % 16)
    bidx = jnp.arange(B)[:, None, None]
    neighborhood = x[bidx, idx]
    neighborhood = neighborhood - new_points[:, :, None, :]
    rep = jnp.broadcast_to(new_points[:, :, None, :], (B, G, K, C))
    neighborhood = jnp.concatenate([neighborhood, rep], axis=-1)

    dcc = _sqdist(center, center)
    ar = jnp.arange(G)
    dcc = dcc.at[:, ar, ar].set(jnp.inf)
    dcc = dcc.at[:, :, 0].set(jnp.inf)
    sel = _chain_pallas(dcc)
    si = (jnp.arange(B)[:, None] * G + sel).reshape(-1)

    neighborhood = neighborhood.reshape(B * G, K, 2 * C)[si].reshape(B, G, K, 2 * C)
    center = center.reshape(B * G, 3)[si].reshape(B, G, 3)
    return (neighborhood, center)


# speedup vs baseline: 13.0227x; 1.2994x over previous
import functools

import jax
import jax.numpy as jnp
from jax.experimental import pallas as pl
from jax.experimental.pallas import tpu as pltpu

_G = 256
_K = 32


def _fps_body(xt_ref, lane_ref, glane_ref, out_ref, *, B, N, G):
    xs = xt_ref[:, 0, :]
    ys = xt_ref[:, 1, :]
    zs = xt_ref[:, 2, :]
    lane = lane_ref[...]
    glane = glane_ref[...]

    def step(i, carry):
        dist, far, sel = carry
        sel = jnp.where(glane == i, far, sel)
        oh = lane == far
        cx = jnp.sum(jnp.where(oh, xs, 0.0), axis=1, keepdims=True)
        cy = jnp.sum(jnp.where(oh, ys, 0.0), axis=1, keepdims=True)
        cz = jnp.sum(jnp.where(oh, zs, 0.0), axis=1, keepdims=True)
        dx = xs - cx
        dy = ys - cy
        dz = zs - cz
        d = (dx * dx + dy * dy) + dz * dz
        dist = jnp.minimum(dist, d)
        m = jnp.max(dist, axis=1, keepdims=True)
        far_new = jnp.min(jnp.where(dist == m, lane, N), axis=1, keepdims=True)
        return dist, far_new, sel

    dist0 = jnp.full((B, N), jnp.inf, dtype=jnp.float32)
    far0 = jnp.zeros((B, 1), dtype=jnp.int32)
    sel0 = jnp.zeros((B, G), dtype=jnp.int32)
    _, _, sel = jax.lax.fori_loop(0, G, step, (dist0, far0, sel0))
    out_ref[...] = sel


def _fps_pallas(xyz):
    B, N, _ = xyz.shape
    xt = jnp.transpose(xyz, (0, 2, 1))
    lane = jnp.broadcast_to(jnp.arange(N, dtype=jnp.int32)[None, :], (B, N))
    glane = jnp.broadcast_to(jnp.arange(_G, dtype=jnp.int32)[None, :], (B, _G))
    return pl.pallas_call(
        functools.partial(_fps_body, B=B, N=N, G=_G),
        out_shape=jax.ShapeDtypeStruct((B, _G), jnp.int32),
    )(xt, lane, glane)


def _chain_body(d_ref, glane_ref, out_ref, *, B, G):
    glane = glane_ref[...]

    def step(i, carry):
        visited, sel, *lasts = carry
        rows = jnp.concatenate(
            [d_ref[b, pl.ds(lasts[b], 1), :] for b in range(B)], axis=0)
        rows = jnp.where(visited != 0, jnp.inf, rows)
        m = jnp.min(rows, axis=1, keepdims=True)
        nxt = jnp.min(jnp.where(rows == m, glane, G), axis=1, keepdims=True)
        visited = visited | (glane == nxt).astype(jnp.int32)
        sel = jnp.where(glane == i, nxt, sel)
        new_lasts = [jnp.sum(nxt[b:b + 1, :]) for b in range(B)]
        return (visited, sel, *new_lasts)

    visited0 = jnp.zeros((B, G), dtype=jnp.int32)
    sel0 = jnp.zeros((B, G), dtype=jnp.int32)
    lasts0 = [jnp.int32(0)] * B
    out = jax.lax.fori_loop(1, G, step, (visited0, sel0, *lasts0))
    out_ref[...] = out[1]


def _chain_pallas(dcc):
    B, G, _ = dcc.shape
    glane = jnp.broadcast_to(jnp.arange(G, dtype=jnp.int32)[None, :], (B, G))
    return pl.pallas_call(
        functools.partial(_chain_body, B=B, G=G),
        out_shape=jax.ShapeDtypeStruct((B, G), jnp.int32),
    )(dcc, glane)


def _sqdist(a, b):
    a2 = jnp.sum(a * a, axis=-1)[:, :, None]
    b2 = jnp.sum(b * b, axis=-1)[:, None, :]
    return a2 + b2 - 2.0 * jnp.einsum('bmd,bnd->bmn', a, b)


def kernel(x, xyz):
    B, N, C = x.shape
    G, K = _G, _K

    fps_idx = _fps_pallas(xyz)
    center = jnp.take_along_axis(xyz, fps_idx[:, :, None], axis=1)
    new_points = jnp.take_along_axis(x, fps_idx[:, :, None], axis=1)

    d = _sqdist(center, xyz)
    d4 = d.reshape(B, G, N // 16, 16)
    m16 = jnp.min(d4, axis=-1)
    _, cid = jax.lax.top_k(-m16, K)
    cid = jnp.sort(cid, axis=-1)
    cand = jnp.take_along_axis(d4, cid[..., None], axis=2)
    _, p = jax.lax.top_k(-cand.reshape(B, G, 16 * K), K)
    cidx = jnp.take_along_axis(cid, p // 16, axis=-1)
    idx = cidx * 16 + (p % 16)
    bidx = jnp.arange(B)[:, None, None]
    neighborhood = x[bidx, idx]
    neighborhood = neighborhood - new_points[:, :, None, :]
    rep = jnp.broadcast_to(new_points[:, :, None, :], (B, G, K, C))
    neighborhood = jnp.concatenate([neighborhood, rep], axis=-1)

    dcc = _sqdist(center, center)
    ar = jnp.arange(G)
    dcc = dcc.at[:, ar, ar].set(jnp.inf)
    dcc = dcc.at[:, :, 0].set(jnp.inf)
    sel = _chain_pallas(dcc)
    si = (jnp.arange(B)[:, None] * G + sel).reshape(-1)

    neighborhood = neighborhood.reshape(B * G, K, 2 * C)[si].reshape(B, G, K, 2 * C)
    center = center.reshape(B * G, 3)[si].reshape(B, G, 3)
    return (neighborhood, center)


# speedup vs baseline: 14.2680x; 1.0956x over previous
import functools

import jax
import jax.numpy as jnp
from jax.experimental import pallas as pl
from jax.experimental.pallas import tpu as pltpu

_G = 256
_K = 32


def _fps_body(xt_ref, lane_ref, glane_ref, out_ref, *, B, N, G):
    xs = xt_ref[:, 0, :]
    ys = xt_ref[:, 1, :]
    zs = xt_ref[:, 2, :]
    lane = lane_ref[...]
    glane = glane_ref[...]

    def step(i, carry):
        dist, far, sel = carry
        sel = jnp.where(glane == i, far, sel)
        oh = lane == far
        cx = jnp.sum(jnp.where(oh, xs, 0.0), axis=1, keepdims=True)
        cy = jnp.sum(jnp.where(oh, ys, 0.0), axis=1, keepdims=True)
        cz = jnp.sum(jnp.where(oh, zs, 0.0), axis=1, keepdims=True)
        dx = xs - cx
        dy = ys - cy
        dz = zs - cz
        d = (dx * dx + dy * dy) + dz * dz
        dist = jnp.minimum(dist, d)
        m = jnp.max(dist, axis=1, keepdims=True)
        far_new = jnp.min(jnp.where(dist == m, lane, N), axis=1, keepdims=True)
        return dist, far_new, sel

    dist0 = jnp.full((B, N), jnp.inf, dtype=jnp.float32)
    far0 = jnp.zeros((B, 1), dtype=jnp.int32)
    sel0 = jnp.zeros((B, G), dtype=jnp.int32)
    _, _, sel = jax.lax.fori_loop(0, G, step, (dist0, far0, sel0))
    out_ref[...] = sel


def _fps_pallas(xyz):
    B, N, _ = xyz.shape
    xt = jnp.transpose(xyz, (0, 2, 1))
    lane = jnp.broadcast_to(jnp.arange(N, dtype=jnp.int32)[None, :], (B, N))
    glane = jnp.broadcast_to(jnp.arange(_G, dtype=jnp.int32)[None, :], (B, _G))
    return pl.pallas_call(
        functools.partial(_fps_body, B=B, N=N, G=_G),
        out_shape=jax.ShapeDtypeStruct((B, _G), jnp.int32),
    )(xt, lane, glane)


def _chain_body(d_ref, glane_ref, out_ref, *, B, G):
    glane = glane_ref[...]

    def step(i, carry):
        visited, sel, *lasts = carry
        rows = jnp.concatenate(
            [d_ref[b, pl.ds(lasts[b], 1), :] for b in range(B)], axis=0)
        rows = jnp.where(visited != 0, jnp.inf, rows)
        m = jnp.min(rows, axis=1, keepdims=True)
        nxt = jnp.min(jnp.where(rows == m, glane, G), axis=1, keepdims=True)
        visited = visited | (glane == nxt).astype(jnp.int32)
        sel = jnp.where(glane == i, nxt, sel)
        new_lasts = [jnp.sum(nxt[b:b + 1, :]) for b in range(B)]
        return (visited, sel, *new_lasts)

    visited0 = jnp.zeros((B, G), dtype=jnp.int32)
    sel0 = jnp.zeros((B, G), dtype=jnp.int32)
    lasts0 = [jnp.int32(0)] * B
    out = jax.lax.fori_loop(1, G, step, (visited0, sel0, *lasts0))
    out_ref[...] = out[1]


def _chain_pallas(dcc):
    B, G, _ = dcc.shape
    glane = jnp.broadcast_to(jnp.arange(G, dtype=jnp.int32)[None, :], (B, G))
    return pl.pallas_call(
        functools.partial(_chain_body, B=B, G=G),
        out_shape=jax.ShapeDtypeStruct((B, G), jnp.int32),
    )(dcc, glane)


def _sqdist(a, b):
    a2 = jnp.sum(a * a, axis=-1)[:, :, None]
    b2 = jnp.sum(b * b, axis=-1)[:, None, :]
    return a2 + b2 - 2.0 * jnp.einsum('bmd,bnd->bmn', a, b)


def kernel(x, xyz):
    B, N, C = x.shape
    G, K = _G, _K

    fps_idx = _fps_pallas(xyz)
    center = jnp.take_along_axis(xyz, fps_idx[:, :, None], axis=1)
    new_points = jnp.take_along_axis(x, fps_idx[:, :, None], axis=1)

    d = _sqdist(center, xyz)
    d4 = d.reshape(B, G, N // 16, 16)
    m16 = jnp.min(d4, axis=-1)
    m256 = jnp.min(m16.reshape(B, G, N // 256, 16), axis=-1)
    _, gid = jax.lax.top_k(-m256, K)
    gid = jnp.sort(gid, axis=-1)
    mcand = jnp.take_along_axis(m16.reshape(B, G, N // 256, 16),
                                gid[..., None], axis=2)
    _, q = jax.lax.top_k(-mcand.reshape(B, G, 16 * K), K)
    cid = jnp.take_along_axis(gid, q // 16, axis=-1) * 16 + (q % 16)
    cid = jnp.sort(cid, axis=-1)
    cand = jnp.take_along_axis(d4, cid[..., None], axis=2)
    _, p = jax.lax.top_k(-cand.reshape(B, G, 16 * K), K)
    cidx = jnp.take_along_axis(cid, p // 16, axis=-1)
    idx = cidx * 16 + (p % 16)

    dcc = _sqdist(center, center)
    ar = jnp.arange(G)
    dcc = dcc.at[:, ar, ar].set(jnp.inf)
    dcc = dcc.at[:, :, 0].set(jnp.inf)
    sel = _chain_pallas(dcc)

    idx_p = jnp.take_along_axis(idx, sel[:, :, None], axis=1)
    np_p = jnp.take_along_axis(new_points, sel[:, :, None], axis=1)
    center_p = jnp.take_along_axis(center, sel[:, :, None], axis=1)
    bidx = jnp.arange(B)[:, None, None]
    neighborhood = x[bidx, idx_p]
    neighborhood = neighborhood - np_p[:, :, None, :]
    rep = jnp.broadcast_to(np_p[:, :, None, :], (B, G, K, C))
    neighborhood = jnp.concatenate([neighborhood, rep], axis=-1)
    return (neighborhood, center_p)


# speedup vs baseline: 14.2704x; 1.0002x over previous
import functools

import jax
import jax.numpy as jnp
from jax.experimental import pallas as pl

_G = 256
_K = 32


def _fps_body(xt_ref, lane_ref, glane_ref, out_ref, *, B, N, G):
    xs = xt_ref[:, 0, :]
    ys = xt_ref[:, 1, :]
    zs = xt_ref[:, 2, :]
    lane = lane_ref[...]
    glane = glane_ref[...]

    def step(i, carry):
        dist, far, sel = carry
        sel = jnp.where(glane == i, far, sel)
        oh = lane == far
        cx = jnp.sum(jnp.where(oh, xs, 0.0), axis=1, keepdims=True)
        cy = jnp.sum(jnp.where(oh, ys, 0.0), axis=1, keepdims=True)
        cz = jnp.sum(jnp.where(oh, zs, 0.0), axis=1, keepdims=True)
        dx = xs - cx
        dy = ys - cy
        dz = zs - cz
        d = (dx * dx + dy * dy) + dz * dz
        dist = jnp.minimum(dist, d)
        m = jnp.max(dist, axis=1, keepdims=True)
        far_new = jnp.min(jnp.where(dist == m, lane, N), axis=1, keepdims=True)
        return dist, far_new, sel

    dist0 = jnp.full((B, N), jnp.inf, dtype=jnp.float32)
    far0 = jnp.zeros((B, 1), dtype=jnp.int32)
    sel0 = jnp.zeros((B, G), dtype=jnp.int32)
    _, _, sel = jax.lax.fori_loop(0, G, step, (dist0, far0, sel0))
    out_ref[...] = sel


def _fps_pallas(xyz):
    B, N, _ = xyz.shape
    xt = jnp.transpose(xyz, (0, 2, 1))
    lane = jnp.broadcast_to(jnp.arange(N, dtype=jnp.int32)[None, :], (B, N))
    glane = jnp.broadcast_to(jnp.arange(_G, dtype=jnp.int32)[None, :], (B, _G))
    return pl.pallas_call(
        functools.partial(_fps_body, B=B, N=N, G=_G),
        out_shape=jax.ShapeDtypeStruct((B, _G), jnp.int32),
    )(xt, lane, glane)


def _chain_body(d_ref, glane_ref, out_ref, *, B, G):
    glane = glane_ref[...]

    def step(i, carry):
        visited, sel, *lasts = carry
        rows = jnp.concatenate(
            [d_ref[b, pl.ds(lasts[b], 1), :] for b in range(B)], axis=0)
        rows = jnp.where(visited != 0, jnp.inf, rows)
        m = jnp.min(rows, axis=1, keepdims=True)
        nxt = jnp.min(jnp.where(rows == m, glane, G), axis=1, keepdims=True)
        visited = visited | (glane == nxt).astype(jnp.int32)
        sel = jnp.where(glane == i, nxt, sel)
        new_lasts = [jnp.sum(nxt[b:b + 1, :]) for b in range(B)]
        return (visited, sel, *new_lasts)

    visited0 = jnp.zeros((B, G), dtype=jnp.int32)
    sel0 = jnp.zeros((B, G), dtype=jnp.int32)
    lasts0 = [jnp.int32(0)] * B
    out = jax.lax.fori_loop(1, G, step, (visited0, sel0, *lasts0))
    out_ref[...] = out[1]


def _chain_pallas(dcc):
    B, G, _ = dcc.shape
    glane = jnp.broadcast_to(jnp.arange(G, dtype=jnp.int32)[None, :], (B, G))
    return pl.pallas_call(
        functools.partial(_chain_body, B=B, G=G),
        out_shape=jax.ShapeDtypeStruct((B, G), jnp.int32),
    )(dcc, glane)


def _sqdist(a, b):
    a2 = jnp.sum(a * a, axis=-1)[:, :, None]
    b2 = jnp.sum(b * b, axis=-1)[:, None, :]
    return a2 + b2 - 2.0 * jnp.einsum('bmd,bnd->bmn', a, b)


def kernel(x, xyz):
    B, N, C = x.shape
    G, K = _G, _K

    fps_idx = _fps_pallas(xyz)
    center = jnp.take_along_axis(xyz, fps_idx[:, :, None], axis=1)
    new_points = jnp.take_along_axis(x, fps_idx[:, :, None], axis=1)

    d = _sqdist(center, xyz)
    d4 = d.reshape(B, G, N // 16, 16)
    m16 = jnp.min(d4, axis=-1)
    m256 = jnp.min(m16.reshape(B, G, N // 256, 16), axis=-1)
    _, gid = jax.lax.top_k(-m256, K)
    gid = jnp.sort(gid, axis=-1)
    mcand = jnp.take_along_axis(m16.reshape(B, G, N // 256, 16),
                                gid[..., None], axis=2)
    _, q = jax.lax.top_k(-mcand.reshape(B, G, 16 * K), K)
    cid = jnp.take_along_axis(gid, q // 16, axis=-1) * 16 + (q % 16)
    cid = jnp.sort(cid, axis=-1)
    cand = jnp.take_along_axis(d4, cid[..., None], axis=2)
    _, p = jax.lax.top_k(-cand.reshape(B, G, 16 * K), K)
    cidx = jnp.take_along_axis(cid, p // 16, axis=-1)
    idx = cidx * 16 + (p % 16)

    dcc = _sqdist(center, center)
    ar = jnp.arange(G)
    dcc = dcc.at[:, ar, ar].set(jnp.inf)
    dcc = dcc.at[:, :, 0].set(jnp.inf)
    sel = _chain_pallas(dcc)

    idx_p = jnp.take_along_axis(idx, sel[:, :, None], axis=1)
    np_p = jnp.take_along_axis(new_points, sel[:, :, None], axis=1)
    center_p = jnp.take_along_axis(center, sel[:, :, None], axis=1)
    bidx = jnp.arange(B)[:, None, None]
    neighborhood = x[bidx, idx_p]
    neighborhood = neighborhood - np_p[:, :, None, :]
    rep = jnp.broadcast_to(np_p[:, :, None, :], (B, G, K, C))
    neighborhood = jnp.concatenate([neighborhood, rep], axis=-1)
    return (neighborhood, center_p)
